# baseline (speedup 1.0000x reference)
import functools

import jax
import jax.numpy as jnp
from jax import lax
from jax.experimental import pallas as pl
from jax.experimental.pallas import tpu as pltpu
from jax.experimental.pallas import tpu_sc as plsc

N = 10000
K = 16
P = 8
NCH = N // P
NW = 32
TR = 400
NT = N // TR
F32 = jnp.float32


def _sc_gather_rows(CW, tc_tiling=True, nrows=N * K):
  mesh = plsc.VectorSubcoreMesh(core_axis_name="c", subcore_axis_name="s")
  cp = None if tc_tiling else pltpu.CompilerParams(use_tc_tiling_on_sc=False)
  pts = P if CW >= 64 else 40
  ng = pts * K // 128
  nch = nrows // (pts * K)

  @functools.partial(
      pl.kernel,
      out_type=jax.ShapeDtypeStruct((nrows, CW), F32),
      mesh=mesh,
      compiler_params=cp,
      scratch_types=[
          pltpu.VMEM((pts * K,), jnp.int32),
          pltpu.VMEM((pts * K, CW), F32),
          pltpu.SemaphoreType.DMA,
      ],
      name=f"sc_gather_rows_{CW}")
  def k(src_hbm, idx_hbm, out_hbm, idx_v, rows_v, sem):
    wid = lax.axis_index("s") * 2 + lax.axis_index("c")

    def do_chunk(j, carry):
      ch = wid + NW * j

      @pl.when(ch < nch)
      def _():
        base = ch * (pts * K)
        pltpu.sync_copy(idx_hbm.at[pl.ds(base, pts * K)], idx_v)
        descs = [
            pltpu.async_copy(src_hbm.at[idx_v.at[pl.ds(i * 128, 128)]],
                             rows_v.at[pl.ds(i * 128, 128)], sem)
            for i in range(ng)
        ]
        for d in descs:
          d.wait()
        pltpu.sync_copy(rows_v, out_hbm.at[pl.ds(base, pts * K)])
      return carry

    lax.fori_loop(0, (nch + NW - 1) // NW, do_chunk, 0)

  return k


def _lrelu(x):
  return jnp.where(x > 0, x, 0.2 * x)


def _row(spec_cols):
  return pl.BlockSpec((TR, spec_cols), lambda *g: (g[-1], 0))


def _whole(r, c):
  return pl.BlockSpec((r, c), lambda *g: (0, 0))


def _dot(a, b):
  return jnp.dot(a, b, preferred_element_type=F32)


def _dot_hi(a, b):
  return jnp.dot(a, b, preferred_element_type=F32,
                 precision=lax.Precision.HIGHEST)


NH = N // 2
TRE = 200
NTE = NH // TRE


def _tc_edge_p0(CW, O_w, O_j, h):
  OT = O_w + O_j

  def body(nb_ref, x_ref, w1_ref, w2_ref, m_ref, acc_ref):
    t = pl.program_id(0)
    nb = nb_ref[...]
    x = x_ref[...]
    e1 = (nb - x[:, None, :]).reshape(TRE * K, CW)
    a3 = _dot(e1, w1_ref[...]).reshape(TRE, K, OT)
    c = _dot(x, w2_ref[...])
    m_ref[...] = jnp.max(a3, axis=1) + c
    aw = a3[:, :, :O_w]
    cw = c[:, :O_w]
    sk = jnp.sum(aw, axis=1)
    ones_row = jnp.ones((1, TRE * K), F32)
    sqsum = _dot_hi(ones_row, (aw * aw).reshape(TRE * K, O_w))

    @pl.when(t == 0)
    def _():
      acc_ref[...] = jnp.zeros_like(acc_ref)
    acc_ref[0:1] += (jnp.sum(sk, axis=0, keepdims=True)
                     + K * jnp.sum(cw, axis=0, keepdims=True))
    acc_ref[1:2] += (sqsum
                     + jnp.sum(2.0 * cw * sk + K * cw * cw,
                               axis=0, keepdims=True))

  return pl.pallas_call(
      body, grid=(NTE,),
      in_specs=[pl.BlockSpec((TRE, K, CW), lambda t: (t, 0, 0)),
                pl.BlockSpec((TRE, CW), lambda t: (t + h * NTE, 0)),
                pl.BlockSpec((CW, OT), lambda t: (0, 0)),
                pl.BlockSpec((CW, OT), lambda t: (0, 0))],
      out_specs=[pl.BlockSpec((TRE, OT), lambda t: (t, 0)),
                 pl.BlockSpec((2, O_w), lambda t: (0, 0))],
      out_shape=[jax.ShapeDtypeStruct((NH, OT), F32),
                 jax.ShapeDtypeStruct((2, O_w), F32)],
  )


def _tc_edge_p1(O_w, O_j, bn_eps=1e-5):
  OT = O_w + O_j

  def body(m_ref, sa_ref, sb_ref, bj_ref, h_ref):
    cnt = float(N * K)
    mean = (sa_ref[0:1] + sb_ref[0:1]) / cnt
    ey2 = (sa_ref[1:2] + sb_ref[1:2]) / cnt
    rs = lax.rsqrt(ey2 - mean * mean + bn_eps)
    m = m_ref[...]
    h_w = _lrelu((m[:, :O_w] - mean) * rs)
    h_j = _lrelu(m[:, O_w:] + bj_ref[...])
    h_ref[...] = jnp.concatenate([h_w, h_j], axis=1)

  return pl.pallas_call(
      body, grid=(NTE,),
      in_specs=[pl.BlockSpec((TRE, OT), lambda t: (t, 0)),
                pl.BlockSpec((2, O_w), lambda t: (0, 0)),
                pl.BlockSpec((2, O_w), lambda t: (0, 0)),
                pl.BlockSpec((1, O_j), lambda t: (0, 0))],
      out_specs=pl.BlockSpec((TRE, OT), lambda t: (t, 0)),
      out_shape=jax.ShapeDtypeStruct((NH, OT), F32),
  )


def _edge_layer(CW, O_w, O_j, tc_tiling=True):
  gat = _sc_gather_rows(CW, tc_tiling=tc_tiling, nrows=NH * K)
  p0a = _tc_edge_p0(CW, O_w, O_j, 0)
  p0b = _tc_edge_p0(CW, O_w, O_j, 1)
  p1 = _tc_edge_p1(O_w, O_j)

  def f(src, idxflat, w1cat, w2cat, bj):
    nba = gat(src, idxflat[:NH * K]).reshape(NH, K, CW)
    nbb = gat(src, idxflat[NH * K:]).reshape(NH, K, CW)
    ma, sa = p0a(nba, src, w1cat, w2cat)
    mb, sb = p0b(nbb, src, w1cat, w2cat)
    ha = p1(ma, sa, sb, bj)
    hb = p1(mb, sa, sb, bj)
    return jnp.concatenate([ha, hb], axis=0)
  return f


def _tc_gc():
  def body(x0_ref, h1_ref, h2_ref, h3_ref, gw_ref, zg_ref, zs_ref, zq_ref):
    t = pl.program_id(0)
    local = jnp.concatenate(
        [x0_ref[...], h1_ref[...], h2_ref[...], h3_ref[...]], axis=1)
    zg = _dot(local, gw_ref[...])
    zg_ref[...] = zg

    @pl.when(t == 0)
    def _():
      zs_ref[...] = jnp.zeros_like(zs_ref)
      zq_ref[...] = jnp.zeros_like(zq_ref)
    zs_ref[...] += jnp.sum(zg, axis=0, keepdims=True)
    zq_ref[...] += jnp.sum(zg * zg, axis=0, keepdims=True)

  one = lambda c: pl.BlockSpec((1, c), lambda t: (0, 0))
  return pl.pallas_call(
      body, grid=(NT,),
      in_specs=[_row(8), _row(64), _row(128), _row(256), _whole(456, 512)],
      out_specs=[_row(512), one(512), one(512)],
      out_shape=[jax.ShapeDtypeStruct((N, 512), F32),
                 jax.ShapeDtypeStruct((1, 512), F32),
                 jax.ShapeDtypeStruct((1, 512), F32)],
  )


def _tc_gred():
  def body(zg_ref, zs_ref, zq_ref, wg1_ref, wg2_ref,
           gmax_ref, gmean_ref, c1_ref, mx_acc, sm_acc):
    t = pl.program_id(0)

    @pl.when(t == 0)
    def _():
      mx_acc[...] = jnp.full_like(mx_acc, -1e30)
      sm_acc[...] = jnp.zeros_like(sm_acc)

    mean = zs_ref[...] / N
    var = zq_ref[...] / N - mean * mean
    g = _lrelu((zg_ref[...] - mean) * lax.rsqrt(var + 1e-5))
    mx_acc[...] = jnp.maximum(mx_acc[...], jnp.max(g, axis=0, keepdims=True))
    sm_acc[...] += jnp.sum(g, axis=0, keepdims=True)

    @pl.when(t == NT - 1)
    def _():
      gmax = mx_acc[...]
      gmean = sm_acc[...] / N
      gmax_ref[...] = gmax
      gmean_ref[...] = gmean
      c1_ref[...] = _dot(gmax, wg1_ref[...]) + _dot(gmean, wg2_ref[...])

  one = lambda c: pl.BlockSpec((1, c), lambda t: (0, 0))
  return pl.pallas_call(
      body, grid=(NT,),
      in_specs=[_row(512), one(512), one(512),
                _whole(512, 1024), _whole(512, 1024)],
      out_specs=[one(512), one(512), one(1024)],
      out_shape=[jax.ShapeDtypeStruct((1, 512), F32),
                 jax.ShapeDtypeStruct((1, 512), F32),
                 jax.ShapeDtypeStruct((1, 1024), F32)],
      scratch_shapes=[pltpu.VMEM((1, 512), F32), pltpu.VMEM((1, 512), F32)],
  )


def _tc_wm1():
  def body(x0_ref, h1_ref, h2_ref, h3_ref, c1_ref, wl_ref,
           z_ref, zs_ref, zq_ref):
    t = pl.program_id(0)
    local = jnp.concatenate(
        [x0_ref[...], h1_ref[...], h2_ref[...], h3_ref[...]], axis=1)
    z = _dot(local, wl_ref[...]) + c1_ref[...]
    z_ref[...] = z

    @pl.when(t == 0)
    def _():
      zs_ref[...] = jnp.zeros_like(zs_ref)
      zq_ref[...] = jnp.zeros_like(zq_ref)
    zs_ref[...] += jnp.sum(z, axis=0, keepdims=True)
    zq_ref[...] += jnp.sum(z * z, axis=0, keepdims=True)

  one = lambda c: pl.BlockSpec((1, c), lambda t: (0, 0))
  return pl.pallas_call(
      body, grid=(NT,),
      in_specs=[_row(8), _row(64), _row(128), _row(256),
                one(1024), _whole(456, 1024)],
      out_specs=[_row(1024), one(1024), one(1024)],
      out_shape=[jax.ShapeDtypeStruct((N, 1024), F32),
                 jax.ShapeDtypeStruct((1, 1024), F32),
                 jax.ShapeDtypeStruct((1, 1024), F32)],
  )


def _tc_wm_mid(Ci, Co):
  def body(z_ref, zs_ref, zq_ref, w_ref, z2_ref, z2s_ref, z2q_ref):
    t = pl.program_id(0)
    mean = zs_ref[...] / N
    var = zq_ref[...] / N - mean * mean
    a = _lrelu((z_ref[...] - mean) * lax.rsqrt(var + 1e-5))
    z2 = _dot(a, w_ref[...])
    z2_ref[...] = z2

    @pl.when(t == 0)
    def _():
      z2s_ref[...] = jnp.zeros_like(z2s_ref)
      z2q_ref[...] = jnp.zeros_like(z2q_ref)
    z2s_ref[...] += jnp.sum(z2, axis=0, keepdims=True)
    z2q_ref[...] += jnp.sum(z2 * z2, axis=0, keepdims=True)

  one = lambda c: pl.BlockSpec((1, c), lambda t: (0, 0))
  return pl.pallas_call(
      body, grid=(NT,),
      in_specs=[_row(Ci), one(Ci), one(Ci), _whole(Ci, Co)],
      out_specs=[_row(Co), one(Co), one(Co)],
      out_shape=[jax.ShapeDtypeStruct((N, Co), F32),
                 jax.ShapeDtypeStruct((1, Co), F32),
                 jax.ShapeDtypeStruct((1, Co), F32)],
  )


def _tc_wm4():
  def body(z_ref, zs_ref, zq_ref, w4_ref, b4_ref,
           x0_ref, h1_ref, h2_ref, h3_ref, att_ref, jn_ref):
    t = pl.program_id(0)
    mean = zs_ref[...] / N
    var = zq_ref[...] / N - mean * mean
    a = _lrelu((z_ref[...] - mean) * lax.rsqrt(var + 1e-5))
    att = _dot(a, w4_ref[...]) + b4_ref[...]
    att_ref[...] = att
    rowmax = jnp.max(att, axis=1, keepdims=True)
    wb = (att == rowmax).astype(F32)
    feat = jnp.concatenate(
        [x0_ref[...], h1_ref[...], h2_ref[...], h3_ref[...]], axis=1)
    jn = lax.dot_general(wb, feat, (((0,), (0,)), ((), ())),
                         preferred_element_type=F32)

    @pl.when(t == 0)
    def _():
      jn_ref[...] = jnp.zeros_like(jn_ref)
    jn_ref[...] += jn

  one = lambda c: pl.BlockSpec((1, c), lambda t: (0, 0))
  return pl.pallas_call(
      body, grid=(NT,),
      in_specs=[_row(64), one(64), one(64), _whole(64, 32), one(32),
                _row(8), _row(64), _row(128), _row(256)],
      out_specs=[_row(32), pl.BlockSpec((32, 456), lambda t: (0, 0))],
      out_shape=[jax.ShapeDtypeStruct((N, 32), F32),
                 jax.ShapeDtypeStruct((32, 456), F32)],
  )


def _tc_skel():
  def body(jn_ref, sidx_ref,
           s1w_ref, s1b_ref, s2w_ref, s2b_ref, s3w_ref, s3b_ref,
           m1w_ref, m1b_ref, m2w_ref, m2b_ref, m3w_ref, m3b_ref, out_ref):
    iota_m = lax.broadcasted_iota(jnp.int32, (32, 32), 1)

    def edge_layer(X, wcat, b):
      mx = None
      for kk in range(4):
        sc = sidx_ref[:, kk:kk + 1]
        pk = (sc == iota_m).astype(F32)
        nb = _dot_hi(pk, X)
        e = jnp.concatenate([nb - X, X], axis=1)
        yk = _dot(e, wcat) + b
        mx = yk if mx is None else jnp.maximum(mx, yk)
      return _lrelu(mx)

    jn = jn_ref[...]
    jp = jn * (1.0 / (jn[:, 3:4] + 1e-5))
    v1 = edge_layer(jp, s1w_ref[...], s1b_ref[...])
    v2 = edge_layer(v1, s2w_ref[...], s2b_ref[...])
    v3 = edge_layer(v2, s3w_ref[...], s3b_ref[...])
    jcat = jnp.concatenate([v1, v2, v3], axis=1)
    j1 = _lrelu(_dot(jcat, m1w_ref[...]) + m1b_ref[...])
    j2 = _lrelu(_dot(j1, m2w_ref[...]) + m2b_ref[...])
    out_ref[...] = _dot(j2, m3w_ref[...]) + m3b_ref[...]

  one = lambda c: pl.BlockSpec((1, c), lambda: (0, 0))
  w = lambda r, c: pl.BlockSpec((r, c), lambda: (0, 0))
  return pl.pallas_call(
      body,
      in_specs=[w(32, 456), w(32, 8),
                w(912, 256), one(256),
                w(512, 128), one(128),
                w(256, 64), one(64),
                w(448, 512), one(512), w(512, 256), one(256),
                w(256, 8), one(8)],
      out_specs=w(32, 8),
      out_shape=jax.ShapeDtypeStruct((32, 8), F32),
  )


def _pad_rows(w, rows):
  return jnp.concatenate(
      [w, jnp.zeros((rows - w.shape[0], w.shape[1]), w.dtype)], axis=0)


def _pad_cols(w, cols):
  return jnp.concatenate(
      [w, jnp.zeros((w.shape[0], cols - w.shape[1]), w.dtype)], axis=1)


def _expand451(wt):
  return jnp.concatenate(
      [wt[:3], jnp.zeros((5, wt.shape[1]), wt.dtype), wt[3:]], axis=0)


def _blockdiag(a, b):
  za = jnp.zeros((a.shape[0], b.shape[1]), F32)
  zb = jnp.zeros((b.shape[0], a.shape[1]), F32)
  return jnp.concatenate([jnp.concatenate([a, za], axis=1),
                          jnp.concatenate([zb, b], axis=1)], axis=0)


def kernel(V, params, facesOneRingIdx, skeletonOneRingIdx):
  p = params
  x0 = V[0].astype(F32)
  x0p = jnp.concatenate(
      [x0, jnp.ones((N, 1), F32), jnp.zeros((N, 4), F32)], axis=1)
  idxflat = facesOneRingIdx[0].astype(jnp.int32).reshape(N * K)

  def ew(name):
    w = p[name]
    c = w.shape[1] // 2
    return w[:, :c].T, w[:, c:].T

  w1w, w2w = ew('wn_g0_w')
  w1j, w2j = ew('jn_g0_w')
  wcat1 = _pad_rows(jnp.concatenate([w1w, w1j], axis=1), 8)
  wcat2 = _pad_rows(jnp.concatenate([w2w, w2j], axis=1), 8)
  hh1 = _edge_layer(8, 64, 64, tc_tiling=False)(
      x0p, idxflat, wcat1, wcat2, p['jn_g0_b'][None])

  w1w, w2w = ew('wn_g1_w')
  w1j, w2j = ew('jn_g1_w')
  hh2 = _edge_layer(128, 128, 128)(
      hh1, idxflat, _blockdiag(w1w, w1j), _blockdiag(w2w, w2j),
      p['jn_g1_b'][None])

  w1w, w2w = ew('wn_g2_w')
  w1j, w2j = ew('jn_g2_w')
  hh3 = _edge_layer(256, 256, 256)(
      hh2, idxflat, _blockdiag(w1w, w1j), _blockdiag(w2w, w2j),
      p['jn_g2_b'][None])

  h1, jh1 = hh1[:, :64], hh1[:, 64:]
  h2, jh2 = hh2[:, :128], hh2[:, 128:]
  h3, jh3 = hh3[:, :256], hh3[:, 256:]

  gcw = _expand451(p['gc_w'].T)
  zg, zs, zq = _tc_gc()(x0p, h1, h2, h3, gcw)
  gmax, gmean, c1 = _tc_gred()(zg, zs, zq,
                               p['wm1_w'][:, :512].T,
                               p['wm1_w'][:, 512:1024].T)
  wl = _expand451(p['wm1_w'][:, 1024:].T)
  z1, z1s, z1q = _tc_wm1()(x0p, h1, h2, h3, c1, wl)
  z2, z2s, z2q = _tc_wm_mid(1024, 256)(z1, z1s, z1q, p['wm2_w'].T)
  z3, z3s, z3q = _tc_wm_mid(256, 64)(z2, z2s, z2q, p['wm3_w'].T)

  w4p = jnp.concatenate([p['wm4_w'].T, jnp.zeros((64, 8), F32)], axis=1)
  b4p = jnp.concatenate([p['wm4_b'], jnp.full((8,), -1e9, F32)])[None]
  attp, jn_acc = _tc_wm4()(z3, z3s, z3q, w4p, b4p, x0p, jh1, jh2, jh3)
  att = attp[:, :24].T[None]

  sidx = skeletonOneRingIdx[0].astype(jnp.int32)
  sidxp = jnp.pad(sidx, ((0, 8), (0, 4)))
  s1w = jnp.concatenate([_expand451(p['sk1_w'][:, :451].T),
                         _expand451(p['sk1_w'][:, 451:].T)], axis=0)
  s2w = p['sk2_w'].T
  s3w = p['sk3_w'].T
  jm3w = jnp.concatenate([p['jm3_w'].T, jnp.zeros((256, 5), F32)], axis=1)
  jm3b = jnp.concatenate([p['jm3_b'], jnp.zeros((5,), F32)])[None]
  joints_p = _tc_skel()(
      jn_acc, sidxp,
      s1w, p['sk1_b'][None], s2w, p['sk2_b'][None], s3w, p['sk3_b'][None],
      p['jm1_w'].T, p['jm1_b'][None], p['jm2_w'].T, p['jm2_b'][None],
      jm3w, jm3b)
  joints = joints_p[:24, :3][None]
  return joints, att


# speedup vs baseline: 1.1173x; 1.1173x over previous
import functools

import jax
import jax.numpy as jnp
from jax import lax
from jax.experimental import pallas as pl
from jax.experimental.pallas import tpu as pltpu
from jax.experimental.pallas import tpu_sc as plsc

N = 10000
K = 16
P = 8
NCH = N // P
NW = 32
TR = 400
NT = N // TR
F32 = jnp.float32


def _sc_gather_rows(CW, tc_tiling=True, nrows=N * K):
  mesh = plsc.VectorSubcoreMesh(core_axis_name="c", subcore_axis_name="s")
  cp = None if tc_tiling else pltpu.CompilerParams(use_tc_tiling_on_sc=False)
  pts = P if CW >= 64 else 40
  ng = pts * K // 128
  nch = nrows // (pts * K)

  @functools.partial(
      pl.kernel,
      out_type=jax.ShapeDtypeStruct((nrows, CW), F32),
      mesh=mesh,
      compiler_params=cp,
      scratch_types=[
          pltpu.VMEM((pts * K,), jnp.int32),
          pltpu.VMEM((pts * K, CW), F32),
          pltpu.SemaphoreType.DMA,
      ],
      name=f"sc_gather_rows_{CW}")
  def k(src_hbm, idx_hbm, out_hbm, idx_v, rows_v, sem):
    wid = lax.axis_index("s") * 2 + lax.axis_index("c")

    def do_chunk(j, carry):
      ch = wid + NW * j

      @pl.when(ch < nch)
      def _():
        base = ch * (pts * K)
        pltpu.sync_copy(idx_hbm.at[pl.ds(base, pts * K)], idx_v)
        descs = [
            pltpu.async_copy(src_hbm.at[idx_v.at[pl.ds(i * 128, 128)]],
                             rows_v.at[pl.ds(i * 128, 128)], sem)
            for i in range(ng)
        ]
        for d in descs:
          d.wait()
        pltpu.sync_copy(rows_v, out_hbm.at[pl.ds(base, pts * K)])
      return carry

    lax.fori_loop(0, (nch + NW - 1) // NW, do_chunk, 0)

  return k


def _lrelu(x):
  return jnp.where(x > 0, x, 0.2 * x)


def _row(spec_cols):
  return pl.BlockSpec((TR, spec_cols), lambda *g: (g[-1], 0))


def _whole(r, c):
  return pl.BlockSpec((r, c), lambda *g: (0, 0))


def _dot(a, b):
  return jnp.dot(a, b, preferred_element_type=F32)


def _dot_hi(a, b):
  return jnp.dot(a, b, preferred_element_type=F32,
                 precision=lax.Precision.HIGHEST)


NH = N // 2
TRE = 200
NTE = NH // TRE


def _tc_edge_p0(CW, O_w, O_j, h):
  OT = O_w + O_j

  def body(nb_ref, x_ref, w1_ref, w2_ref, m_ref, acc_ref):
    t = pl.program_id(0)
    nb = nb_ref[...]
    x = x_ref[...]
    e1 = (nb - x[:, None, :]).reshape(TRE * K, CW)
    a3 = _dot(e1, w1_ref[...]).reshape(TRE, K, OT)
    c = _dot(x, w2_ref[...])
    m_ref[...] = jnp.max(a3, axis=1) + c
    aw = a3[:, :, :O_w]
    cw = c[:, :O_w]
    sk = jnp.sum(aw, axis=1)

    @pl.when(t == 0)
    def _():
      acc_ref[...] = jnp.zeros_like(acc_ref)
    acc_ref[0:1] += (jnp.sum(sk, axis=0, keepdims=True)
                     + K * jnp.sum(cw, axis=0, keepdims=True))
    acc_ref[1:2] += (jnp.sum(jnp.sum(aw * aw, axis=1), axis=0, keepdims=True)
                     + jnp.sum(2.0 * cw * sk + K * cw * cw,
                               axis=0, keepdims=True))

  return pl.pallas_call(
      body, grid=(NTE,),
      in_specs=[pl.BlockSpec((TRE, K, CW), lambda t: (t, 0, 0)),
                pl.BlockSpec((TRE, CW), lambda t: (t + h * NTE, 0)),
                pl.BlockSpec((CW, OT), lambda t: (0, 0)),
                pl.BlockSpec((CW, OT), lambda t: (0, 0))],
      out_specs=[pl.BlockSpec((TRE, OT), lambda t: (t, 0)),
                 pl.BlockSpec((2, O_w), lambda t: (0, 0))],
      out_shape=[jax.ShapeDtypeStruct((NH, OT), F32),
                 jax.ShapeDtypeStruct((2, O_w), F32)],
  )


def _tc_edge_p1(O_w, O_j, bn_eps=1e-5):
  OT = O_w + O_j

  def body(m_ref, sa_ref, sb_ref, bj_ref, h_ref):
    cnt = float(N * K)
    mean = (sa_ref[0:1] + sb_ref[0:1]) / cnt
    ey2 = (sa_ref[1:2] + sb_ref[1:2]) / cnt
    rs = lax.rsqrt(ey2 - mean * mean + bn_eps)
    m = m_ref[...]
    h_w = _lrelu((m[:, :O_w] - mean) * rs)
    h_j = _lrelu(m[:, O_w:] + bj_ref[...])
    h_ref[...] = jnp.concatenate([h_w, h_j], axis=1)

  return pl.pallas_call(
      body, grid=(NTE,),
      in_specs=[pl.BlockSpec((TRE, OT), lambda t: (t, 0)),
                pl.BlockSpec((2, O_w), lambda t: (0, 0)),
                pl.BlockSpec((2, O_w), lambda t: (0, 0)),
                pl.BlockSpec((1, O_j), lambda t: (0, 0))],
      out_specs=pl.BlockSpec((TRE, OT), lambda t: (t, 0)),
      out_shape=jax.ShapeDtypeStruct((NH, OT), F32),
  )


def _edge_layer(CW, O_w, O_j, tc_tiling=True):
  gat = _sc_gather_rows(CW, tc_tiling=tc_tiling, nrows=NH * K)
  p0a = _tc_edge_p0(CW, O_w, O_j, 0)
  p0b = _tc_edge_p0(CW, O_w, O_j, 1)
  p1 = _tc_edge_p1(O_w, O_j)

  def f(src, idxflat, w1cat, w2cat, bj):
    nba = gat(src, idxflat[:NH * K]).reshape(NH, K, CW)
    nbb = gat(src, idxflat[NH * K:]).reshape(NH, K, CW)
    ma, sa = p0a(nba, src, w1cat, w2cat)
    mb, sb = p0b(nbb, src, w1cat, w2cat)
    ha = p1(ma, sa, sb, bj)
    hb = p1(mb, sa, sb, bj)
    return jnp.concatenate([ha, hb], axis=0)
  return f


def _tc_gc():
  def body(x0_ref, h1_ref, h2_ref, h3_ref, gw_ref, zg_ref, zs_ref, zq_ref):
    t = pl.program_id(0)
    local = jnp.concatenate(
        [x0_ref[...], h1_ref[...], h2_ref[...], h3_ref[...]], axis=1)
    zg = _dot(local, gw_ref[...])
    zg_ref[...] = zg

    @pl.when(t == 0)
    def _():
      zs_ref[...] = jnp.zeros_like(zs_ref)
      zq_ref[...] = jnp.zeros_like(zq_ref)
    zs_ref[...] += jnp.sum(zg, axis=0, keepdims=True)
    zq_ref[...] += jnp.sum(zg * zg, axis=0, keepdims=True)

  one = lambda c: pl.BlockSpec((1, c), lambda t: (0, 0))
  return pl.pallas_call(
      body, grid=(NT,),
      in_specs=[_row(8), _row(64), _row(128), _row(256), _whole(456, 512)],
      out_specs=[_row(512), one(512), one(512)],
      out_shape=[jax.ShapeDtypeStruct((N, 512), F32),
                 jax.ShapeDtypeStruct((1, 512), F32),
                 jax.ShapeDtypeStruct((1, 512), F32)],
  )


def _tc_gred():
  def body(zg_ref, zs_ref, zq_ref, wg1_ref, wg2_ref,
           gmax_ref, gmean_ref, c1_ref, mx_acc, sm_acc):
    t = pl.program_id(0)

    @pl.when(t == 0)
    def _():
      mx_acc[...] = jnp.full_like(mx_acc, -1e30)
      sm_acc[...] = jnp.zeros_like(sm_acc)

    mean = zs_ref[...] / N
    var = zq_ref[...] / N - mean * mean
    g = _lrelu((zg_ref[...] - mean) * lax.rsqrt(var + 1e-5))
    mx_acc[...] = jnp.maximum(mx_acc[...], jnp.max(g, axis=0, keepdims=True))
    sm_acc[...] += jnp.sum(g, axis=0, keepdims=True)

    @pl.when(t == NT - 1)
    def _():
      gmax = mx_acc[...]
      gmean = sm_acc[...] / N
      gmax_ref[...] = gmax
      gmean_ref[...] = gmean
      c1_ref[...] = _dot(gmax, wg1_ref[...]) + _dot(gmean, wg2_ref[...])

  one = lambda c: pl.BlockSpec((1, c), lambda t: (0, 0))
  return pl.pallas_call(
      body, grid=(NT,),
      in_specs=[_row(512), one(512), one(512),
                _whole(512, 1024), _whole(512, 1024)],
      out_specs=[one(512), one(512), one(1024)],
      out_shape=[jax.ShapeDtypeStruct((1, 512), F32),
                 jax.ShapeDtypeStruct((1, 512), F32),
                 jax.ShapeDtypeStruct((1, 1024), F32)],
      scratch_shapes=[pltpu.VMEM((1, 512), F32), pltpu.VMEM((1, 512), F32)],
  )


def _tc_wm1():
  def body(x0_ref, h1_ref, h2_ref, h3_ref, c1_ref, wl_ref,
           z_ref, zs_ref, zq_ref):
    t = pl.program_id(0)
    local = jnp.concatenate(
        [x0_ref[...], h1_ref[...], h2_ref[...], h3_ref[...]], axis=1)
    z = _dot(local, wl_ref[...]) + c1_ref[...]
    z_ref[...] = z

    @pl.when(t == 0)
    def _():
      zs_ref[...] = jnp.zeros_like(zs_ref)
      zq_ref[...] = jnp.zeros_like(zq_ref)
    zs_ref[...] += jnp.sum(z, axis=0, keepdims=True)
    zq_ref[...] += jnp.sum(z * z, axis=0, keepdims=True)

  one = lambda c: pl.BlockSpec((1, c), lambda t: (0, 0))
  return pl.pallas_call(
      body, grid=(NT,),
      in_specs=[_row(8), _row(64), _row(128), _row(256),
                one(1024), _whole(456, 1024)],
      out_specs=[_row(1024), one(1024), one(1024)],
      out_shape=[jax.ShapeDtypeStruct((N, 1024), F32),
                 jax.ShapeDtypeStruct((1, 1024), F32),
                 jax.ShapeDtypeStruct((1, 1024), F32)],
  )


def _tc_wm_mid(Ci, Co):
  def body(z_ref, zs_ref, zq_ref, w_ref, z2_ref, z2s_ref, z2q_ref):
    t = pl.program_id(0)
    mean = zs_ref[...] / N
    var = zq_ref[...] / N - mean * mean
    a = _lrelu((z_ref[...] - mean) * lax.rsqrt(var + 1e-5))
    z2 = _dot(a, w_ref[...])
    z2_ref[...] = z2

    @pl.when(t == 0)
    def _():
      z2s_ref[...] = jnp.zeros_like(z2s_ref)
      z2q_ref[...] = jnp.zeros_like(z2q_ref)
    z2s_ref[...] += jnp.sum(z2, axis=0, keepdims=True)
    z2q_ref[...] += jnp.sum(z2 * z2, axis=0, keepdims=True)

  one = lambda c: pl.BlockSpec((1, c), lambda t: (0, 0))
  return pl.pallas_call(
      body, grid=(NT,),
      in_specs=[_row(Ci), one(Ci), one(Ci), _whole(Ci, Co)],
      out_specs=[_row(Co), one(Co), one(Co)],
      out_shape=[jax.ShapeDtypeStruct((N, Co), F32),
                 jax.ShapeDtypeStruct((1, Co), F32),
                 jax.ShapeDtypeStruct((1, Co), F32)],
  )


def _tc_wm4():
  def body(z_ref, zs_ref, zq_ref, w4_ref, b4_ref,
           x0_ref, h1_ref, h2_ref, h3_ref, att_ref, jn_ref):
    t = pl.program_id(0)
    mean = zs_ref[...] / N
    var = zq_ref[...] / N - mean * mean
    a = _lrelu((z_ref[...] - mean) * lax.rsqrt(var + 1e-5))
    att = _dot(a, w4_ref[...]) + b4_ref[...]
    att_ref[...] = att
    rowmax = jnp.max(att, axis=1, keepdims=True)
    wb = (att == rowmax).astype(F32)
    feat = jnp.concatenate(
        [x0_ref[...], h1_ref[...], h2_ref[...], h3_ref[...]], axis=1)
    jn = lax.dot_general(wb, feat, (((0,), (0,)), ((), ())),
                         preferred_element_type=F32)

    @pl.when(t == 0)
    def _():
      jn_ref[...] = jnp.zeros_like(jn_ref)
    jn_ref[...] += jn

  one = lambda c: pl.BlockSpec((1, c), lambda t: (0, 0))
  return pl.pallas_call(
      body, grid=(NT,),
      in_specs=[_row(64), one(64), one(64), _whole(64, 32), one(32),
                _row(8), _row(64), _row(128), _row(256)],
      out_specs=[_row(32), pl.BlockSpec((32, 456), lambda t: (0, 0))],
      out_shape=[jax.ShapeDtypeStruct((N, 32), F32),
                 jax.ShapeDtypeStruct((32, 456), F32)],
  )


def _tc_skel():
  def body(jn_ref, sidx_ref,
           s1w_ref, s1b_ref, s2w_ref, s2b_ref, s3w_ref, s3b_ref,
           m1w_ref, m1b_ref, m2w_ref, m2b_ref, m3w_ref, m3b_ref, out_ref):
    iota_m = lax.broadcasted_iota(jnp.int32, (32, 32), 1)

    def edge_layer(X, wcat, b):
      mx = None
      for kk in range(4):
        sc = sidx_ref[:, kk:kk + 1]
        pk = (sc == iota_m).astype(F32)
        nb = _dot_hi(pk, X)
        e = jnp.concatenate([nb - X, X], axis=1)
        yk = _dot(e, wcat) + b
        mx = yk if mx is None else jnp.maximum(mx, yk)
      return _lrelu(mx)

    jn = jn_ref[...]
    jp = jn * (1.0 / (jn[:, 3:4] + 1e-5))
    v1 = edge_layer(jp, s1w_ref[...], s1b_ref[...])
    v2 = edge_layer(v1, s2w_ref[...], s2b_ref[...])
    v3 = edge_layer(v2, s3w_ref[...], s3b_ref[...])
    jcat = jnp.concatenate([v1, v2, v3], axis=1)
    j1 = _lrelu(_dot(jcat, m1w_ref[...]) + m1b_ref[...])
    j2 = _lrelu(_dot(j1, m2w_ref[...]) + m2b_ref[...])
    out_ref[...] = _dot(j2, m3w_ref[...]) + m3b_ref[...]

  one = lambda c: pl.BlockSpec((1, c), lambda: (0, 0))
  w = lambda r, c: pl.BlockSpec((r, c), lambda: (0, 0))
  return pl.pallas_call(
      body,
      in_specs=[w(32, 456), w(32, 8),
                w(912, 256), one(256),
                w(512, 128), one(128),
                w(256, 64), one(64),
                w(448, 512), one(512), w(512, 256), one(256),
                w(256, 8), one(8)],
      out_specs=w(32, 8),
      out_shape=jax.ShapeDtypeStruct((32, 8), F32),
  )


def _pad_rows(w, rows):
  return jnp.concatenate(
      [w, jnp.zeros((rows - w.shape[0], w.shape[1]), w.dtype)], axis=0)


def _expand451(wt):
  return jnp.concatenate(
      [wt[:3], jnp.zeros((5, wt.shape[1]), wt.dtype), wt[3:]], axis=0)


def _blockdiag(a, b):
  za = jnp.zeros((a.shape[0], b.shape[1]), F32)
  zb = jnp.zeros((b.shape[0], a.shape[1]), F32)
  return jnp.concatenate([jnp.concatenate([a, za], axis=1),
                          jnp.concatenate([zb, b], axis=1)], axis=0)


def kernel(V, params, facesOneRingIdx, skeletonOneRingIdx):
  p = params
  x0 = V[0].astype(F32)
  x0p = jnp.concatenate(
      [x0, jnp.ones((N, 1), F32), jnp.zeros((N, 4), F32)], axis=1)
  idxflat = facesOneRingIdx[0].astype(jnp.int32).reshape(N * K)

  def ew(name):
    w = p[name]
    c = w.shape[1] // 2
    return w[:, :c].T, w[:, c:].T

  w1w, w2w = ew('wn_g0_w')
  w1j, w2j = ew('jn_g0_w')
  wcat1 = _pad_rows(jnp.concatenate([w1w, w1j], axis=1), 8)
  wcat2 = _pad_rows(jnp.concatenate([w2w, w2j], axis=1), 8)
  hh1 = _edge_layer(8, 64, 64, tc_tiling=False)(
      x0p, idxflat, wcat1, wcat2, p['jn_g0_b'][None])

  w1w, w2w = ew('wn_g1_w')
  w1j, w2j = ew('jn_g1_w')
  hh2 = _edge_layer(128, 128, 128)(
      hh1, idxflat, _blockdiag(w1w, w1j), _blockdiag(w2w, w2j),
      p['jn_g1_b'][None])

  w1w, w2w = ew('wn_g2_w')
  w1j, w2j = ew('jn_g2_w')
  hh3 = _edge_layer(256, 256, 256)(
      hh2, idxflat, _blockdiag(w1w, w1j), _blockdiag(w2w, w2j),
      p['jn_g2_b'][None])

  h1, jh1 = hh1[:, :64], hh1[:, 64:]
  h2, jh2 = hh2[:, :128], hh2[:, 128:]
  h3, jh3 = hh3[:, :256], hh3[:, 256:]

  gcw = _expand451(p['gc_w'].T)
  zg, zs, zq = _tc_gc()(x0p, h1, h2, h3, gcw)
  gmax, gmean, c1 = _tc_gred()(zg, zs, zq,
                               p['wm1_w'][:, :512].T,
                               p['wm1_w'][:, 512:1024].T)
  wl = _expand451(p['wm1_w'][:, 1024:].T)
  z1, z1s, z1q = _tc_wm1()(x0p, h1, h2, h3, c1, wl)
  z2, z2s, z2q = _tc_wm_mid(1024, 256)(z1, z1s, z1q, p['wm2_w'].T)
  z3, z3s, z3q = _tc_wm_mid(256, 64)(z2, z2s, z2q, p['wm3_w'].T)

  w4p = jnp.concatenate([p['wm4_w'].T, jnp.zeros((64, 8), F32)], axis=1)
  b4p = jnp.concatenate([p['wm4_b'], jnp.full((8,), -1e9, F32)])[None]
  attp, jn_acc = _tc_wm4()(z3, z3s, z3q, w4p, b4p, x0p, jh1, jh2, jh3)
  att = attp[:, :24].T[None]

  sidx = skeletonOneRingIdx[0].astype(jnp.int32)
  sidxp = jnp.pad(sidx, ((0, 8), (0, 4)))
  s1w = jnp.concatenate([_expand451(p['sk1_w'][:, :451].T),
                         _expand451(p['sk1_w'][:, 451:].T)], axis=0)
  s2w = p['sk2_w'].T
  s3w = p['sk3_w'].T
  jm3w = jnp.concatenate([p['jm3_w'].T, jnp.zeros((256, 5), F32)], axis=1)
  jm3b = jnp.concatenate([p['jm3_b'], jnp.zeros((5,), F32)])[None]
  joints_p = _tc_skel()(
      jn_acc, sidxp,
      s1w, p['sk1_b'][None], s2w, p['sk2_b'][None], s3w, p['sk3_b'][None],
      p['jm1_w'].T, p['jm1_b'][None], p['jm2_w'].T, p['jm2_b'][None],
      jm3w, jm3b)
  joints = joints_p[:24, :3][None]
  return joints, att
